# Initial kernel scaffold; baseline (speedup 1.0000x reference)
import jax
import jax.numpy as jnp
from jax.experimental import pallas as pl


def kernel(x_eval, x, control_points):
    raise NotImplementedError("write your pallas kernel here")


# speedup vs baseline: 144.6361x; 144.6361x over previous
import dataclasses
import functools

import jax
import jax.numpy as jnp
from jax import lax
from jax.experimental import pallas as pl
from jax.experimental.pallas import tpu as pltpu
from jax.experimental.pallas import tpu_sc as plsc

N_SEG = 65536
N_EV = 2097152
L = 16
NC, NS = 2, 16
NW = NC * NS
PER_W = N_EV // NW
K = 128
NCH = PER_W // K
NV = K // L
XPAD = 65544


def _vfull(v, dtype=jnp.int32):
    return jnp.full((L,), v, dtype)


@jax.jit
def _sc_bezier(x_eval, xpad, table):
    mesh = plsc.VectorSubcoreMesh(core_axis_name="c", subcore_axis_name="s")
    cparams = pltpu.CompilerParams()
    if "needs_layout_passes" in pltpu.CompilerParams.__dataclass_fields__:
        cparams = dataclasses.replace(cparams, needs_layout_passes=False)
    if "use_tc_tiling_on_sc" in pltpu.CompilerParams.__dataclass_fields__:
        cparams = dataclasses.replace(cparams, use_tc_tiling_on_sc=False)

    @functools.partial(
        pl.kernel,
        compiler_params=cparams,
        out_type=(
            jax.ShapeDtypeStruct((N_EV, 3), jnp.float32),
            jax.ShapeDtypeStruct((N_EV,), jnp.int32),
        ),
        mesh=mesh,
        scratch_types=[
            pltpu.VMEM((XPAD,), jnp.float32),
            pltpu.VMEM((K,), jnp.float32),
            pltpu.VMEM((K,), jnp.float32),
            pltpu.VMEM((K,), jnp.int32),
            pltpu.VMEM((K, 16), jnp.float32),
            pltpu.VMEM((K, 3), jnp.float32),
            pltpu.SemaphoreType.DMA,
        ],
    )
    def kern(xev_hbm, x_hbm, tab_hbm, out_hbm, idx_hbm,
             xtab, tin, sbuf, idxv, rows, evout, sem):
        wid = lax.axis_index("s") * NC + lax.axis_index("c")
        pltpu.async_copy(x_hbm, xtab, sem).wait()
        iota = lax.iota(jnp.int32, L)
        spanv = plsc.load_gather(xtab, [_vfull(N_SEG)])
        capv = _vfull(N_SEG - 1)

        @pl.loop(0, NCH)
        def _(c):
            base = wid * PER_W + c * K
            pltpu.async_copy(xev_hbm.at[pl.ds(base, K)], tin, sem).wait()

            @pl.loop(0, NV)
            def _(v):
                t = lax.rem(tin[pl.ds(v * L, L)], spanv)
                lo = _vfull(0)
                for p in (32768, 16384, 8192, 4096, 2048, 1024, 512, 256,
                          128, 64, 32, 16, 8, 4, 2, 1):
                    cand = lax.min(lo + p, capv)
                    xv = plsc.load_gather(xtab, [cand])
                    lo = lax.select(xv <= t, cand, lo)
                xs = plsc.load_gather(xtab, [lo])
                xe = plsc.load_gather(xtab, [lo + 1])
                idxv[pl.ds(v * L, L)] = lo
                sbuf[pl.ds(v * L, L)] = (t - xs) / (xe - xs)

            pltpu.async_copy(tab_hbm.at[idxv], rows, sem).wait()

            @pl.loop(0, NV)
            def _(v):
                s = sbuf[pl.ds(v * L, L)]
                om = 1.0 - s
                s2 = s * s
                om2 = om * om
                b0 = om2 * om
                b1 = (3.0 * s) * om2
                b2 = (3.0 * s2) * om
                b3 = s2 * s
                rid = jnp.full((L,), v * L, jnp.int32) + iota
                for d in range(3):
                    p0 = plsc.load_gather(rows, [rid, _vfull(d)])
                    p1 = plsc.load_gather(rows, [rid, _vfull(3 + d)])
                    p2 = plsc.load_gather(rows, [rid, _vfull(6 + d)])
                    p3 = plsc.load_gather(rows, [rid, _vfull(9 + d)])
                    acc = ((b0 * p0 + b1 * p1) + b2 * p2) + b3 * p3
                    plsc.store_scatter(evout, [rid, _vfull(d)], acc)

            pltpu.async_copy(evout, out_hbm.at[pl.ds(base, K)], sem).wait()
            pltpu.async_copy(idxv, idx_hbm.at[pl.ds(base, K)], sem).wait()

    return kern(x_eval, xpad, table)


def kernel(x_eval, x, control_points):
    xpad = jnp.concatenate(
        [x, jnp.full((XPAD - (N_SEG + 1),), x[-1], x.dtype)])
    table = jnp.concatenate(
        [control_points.reshape(N_SEG, 12),
         jnp.zeros((N_SEG, 4), jnp.float32)], axis=1)
    ev, idx = _sc_bezier(x_eval.reshape(-1), xpad, table)
    return ev.reshape(tuple(x_eval.shape) + (3,)), idx.reshape(x_eval.shape)


# speedup vs baseline: 175.1249x; 1.2108x over previous
import dataclasses
import functools

import jax
import jax.numpy as jnp
from jax import lax
from jax.experimental import pallas as pl
from jax.experimental.pallas import tpu as pltpu
from jax.experimental.pallas import tpu_sc as plsc

N_SEG = 65536
N_EV = 2097152
L = 16
NC, NS = 2, 16
NW = NC * NS
PER_W = N_EV // NW
K = 128
NCH = PER_W // K
NV = K // L
XPAD = 65544


def _vfull(v, dtype=jnp.int32):
    return jnp.full((L,), v, dtype)


@jax.jit
def _sc_bezier(x_eval, xpad, table):
    mesh = plsc.VectorSubcoreMesh(core_axis_name="c", subcore_axis_name="s")
    cparams = pltpu.CompilerParams()
    if "needs_layout_passes" in pltpu.CompilerParams.__dataclass_fields__:
        cparams = dataclasses.replace(cparams, needs_layout_passes=False)
    if "use_tc_tiling_on_sc" in pltpu.CompilerParams.__dataclass_fields__:
        cparams = dataclasses.replace(cparams, use_tc_tiling_on_sc=False)

    @functools.partial(
        pl.kernel,
        compiler_params=cparams,
        out_type=(
            jax.ShapeDtypeStruct((N_EV, 3), jnp.float32),
            jax.ShapeDtypeStruct((N_EV,), jnp.int32),
        ),
        mesh=mesh,
        scratch_types=[
            pltpu.VMEM((XPAD,), jnp.float32),
            pltpu.VMEM((K,), jnp.float32),
            pltpu.VMEM((K,), jnp.float32),
            pltpu.VMEM((K,), jnp.int32),
            pltpu.VMEM((K, 16), jnp.float32),
            pltpu.VMEM((K, 3), jnp.float32),
            pltpu.SemaphoreType.DMA,
        ],
    )
    def kern(xev_hbm, x_hbm, tab_hbm, out_hbm, idx_hbm,
             xtab, tin, sbuf, idxv, rows, evout, sem):
        wid = lax.axis_index("s") * NC + lax.axis_index("c")
        pltpu.async_copy(x_hbm, xtab, sem).wait()
        iota = lax.iota(jnp.int32, L)
        spanv = plsc.load_gather(xtab, [_vfull(N_SEG)])
        capv = _vfull(N_SEG - 1)

        @pl.loop(0, NCH)
        def _(c):
            base = wid * PER_W + c * K
            pltpu.async_copy(xev_hbm.at[pl.ds(base, K)], tin, sem).wait()

            @plsc.parallel_loop(0, NV, unroll=4)
            def _(v):
                t = lax.rem(tin[pl.ds(v * L, L)], spanv)
                lo = _vfull(0)
                for p in (32768, 16384, 8192, 4096, 2048, 1024, 512, 256,
                          128, 64, 32, 16, 8, 4, 2, 1):
                    cand = lax.min(lo + p, capv)
                    xv = plsc.load_gather(xtab, [cand])
                    lo = lax.select(xv <= t, cand, lo)
                xs = plsc.load_gather(xtab, [lo])
                xe = plsc.load_gather(xtab, [lo + 1])
                idxv[pl.ds(v * L, L)] = lo
                sbuf[pl.ds(v * L, L)] = (t - xs) / (xe - xs)

            pltpu.async_copy(tab_hbm.at[idxv], rows, sem).wait()

            @plsc.parallel_loop(0, NV, unroll=2)
            def _(v):
                s = sbuf[pl.ds(v * L, L)]
                om = 1.0 - s
                s2 = s * s
                om2 = om * om
                b0 = om2 * om
                b1 = (3.0 * s) * om2
                b2 = (3.0 * s2) * om
                b3 = s2 * s
                rid = jnp.full((L,), v * L, jnp.int32) + iota
                for d in range(3):
                    p0 = plsc.load_gather(rows, [rid, _vfull(d)])
                    p1 = plsc.load_gather(rows, [rid, _vfull(3 + d)])
                    p2 = plsc.load_gather(rows, [rid, _vfull(6 + d)])
                    p3 = plsc.load_gather(rows, [rid, _vfull(9 + d)])
                    acc = ((b0 * p0 + b1 * p1) + b2 * p2) + b3 * p3
                    plsc.store_scatter(evout, [rid, _vfull(d)], acc)

            pltpu.async_copy(evout, out_hbm.at[pl.ds(base, K)], sem).wait()
            pltpu.async_copy(idxv, idx_hbm.at[pl.ds(base, K)], sem).wait()

    return kern(x_eval, xpad, table)


def kernel(x_eval, x, control_points):
    xpad = jnp.concatenate(
        [x, jnp.full((XPAD - (N_SEG + 1),), x[-1], x.dtype)])
    table = jnp.concatenate(
        [control_points.reshape(N_SEG, 12),
         jnp.zeros((N_SEG, 4), jnp.float32)], axis=1)
    ev, idx = _sc_bezier(x_eval.reshape(-1), xpad, table)
    return ev.reshape(tuple(x_eval.shape) + (3,)), idx.reshape(x_eval.shape)
